# Initial kernel scaffold; baseline (speedup 1.0000x reference)
import jax
import jax.numpy as jnp
from jax.experimental import pallas as pl


def kernel(x, edge_index, W_proj, b_proj, W_msg, b_msg, W_next, b_next, gamma, beta, W_out, b_out):
    raise NotImplementedError("write your pallas kernel here")


# speedup vs baseline: 5.2322x; 5.2322x over previous
import functools

import jax
import jax.numpy as jnp
from jax import lax
from jax.experimental import pallas as pl
from jax.experimental.pallas import tpu as pltpu
from jax.experimental.pallas import tpu_sc as plsc

N = 10000
E = 320000
D = 128
H = 128
OUT = 64

NC = 2
NS = 16
NW = NC * NS
L = 16

EPT = E // NW
CH = 80
NCHUNK = EPT // CH

NPAD = 10240
RPT = NPAD // NS

HQ = NPAD // 128
HB = 5000
HSTEPS = E // HB


def _tc_pre_body(x_ref, wp_ref, bp_ref, wms_ref, wmd_ref, bm_ref, dst_ref,
                 h_ref, a_ref, b_ref, cnt_ref):
    h = jnp.dot(x_ref[...], wp_ref[...],
                preferred_element_type=jnp.float32) + bp_ref[...]
    h_ref[...] = h
    a_ref[...] = jnp.dot(h, wms_ref[...], preferred_element_type=jnp.float32)
    b_ref[...] = jnp.dot(h, wmd_ref[...],
                         preferred_element_type=jnp.float32) + bm_ref[...]

    iq = lax.broadcasted_iota(jnp.int32, (HQ, HB), 0)
    ir = lax.broadcasted_iota(jnp.int32, (HB, 128), 1)

    def _hstep(k, cnt):
        d = dst_ref[k, :]
        oq = (iq == (d // 128)[None, :]).astype(jnp.bfloat16)
        orr = (ir == (d % 128)[:, None]).astype(jnp.bfloat16)
        return cnt + jnp.dot(oq, orr, preferred_element_type=jnp.float32)

    cnt_ref[...] = lax.fori_loop(
        0, HSTEPS, _hstep, jnp.zeros((HQ, 128), jnp.float32))


def _tc_pre(x, w_proj, b_proj, wm_src, wm_dst, b_msg, dst):
    return pl.pallas_call(
        _tc_pre_body,
        out_shape=[
            jax.ShapeDtypeStruct((N, H), jnp.float32),
            jax.ShapeDtypeStruct((N, H), jnp.float32),
            jax.ShapeDtypeStruct((N, H), jnp.float32),
            jax.ShapeDtypeStruct((HQ, 128), jnp.float32),
        ],
    )(x, w_proj, b_proj.reshape(1, H), wm_src, wm_dst, b_msg.reshape(1, H),
      dst.reshape(HSTEPS, HB))


_sc_mesh = plsc.VectorSubcoreMesh(
    core_axis_name="c", subcore_axis_name="s", num_cores=NC, num_subcores=NS)


@functools.partial(
    pl.kernel,
    out_type=jax.ShapeDtypeStruct((NC, NPAD, H), jnp.float32),
    mesh=_sc_mesh,
    scratch_types=[
        pltpu.VMEM((CH,), jnp.int32),
        pltpu.VMEM((CH,), jnp.int32),
        pltpu.VMEM((CH, H), jnp.float32),
        pltpu.VMEM((CH, H), jnp.float32),
        pltpu.VMEM((CH, H), jnp.float32),
        pltpu.VMEM_SHARED((NPAD, H), jnp.float32),
        pltpu.SemaphoreType.DMA,
        pltpu.SemaphoreType.DMA,
    ],
)
def _sc_edge(a_hbm, b_hbm, src_hbm, dst_hbm, out_hbm,
             idx_s, idx_d, a_buf, b_buf, m_buf, acc, sem_a, sem_b):
    c = lax.axis_index("c")
    s = lax.axis_index("s")
    wid = c * NS + s
    ebase = wid * EPT

    zeros = jnp.zeros((L,), jnp.float32)

    def _zero_row(r, _):
        for j in range(H // L):
            m_buf[r, pl.ds(j * L, L)] = zeros
        return 0
    lax.fori_loop(0, CH, _zero_row, 0)
    rbase = s * RPT
    for k in range(RPT // CH):
        pltpu.sync_copy(m_buf, acc.at[pl.ds(rbase + k * CH, CH), :])

    plsc.subcore_barrier()

    def _chunk(ci, _):
        base = ebase + ci * CH
        pltpu.sync_copy(src_hbm.at[pl.ds(base, CH)], idx_s)
        pltpu.sync_copy(dst_hbm.at[pl.ds(base, CH)], idx_d)
        cp_a = pltpu.async_copy(a_hbm.at[idx_s], a_buf, sem_a)
        cp_b = pltpu.async_copy(b_hbm.at[idx_d], b_buf, sem_b)
        cp_a.wait()
        cp_b.wait()

        def _row(r, _):
            for j in range(H // L):
                sl = pl.ds(j * L, L)
                m_buf[r, sl] = jnp.maximum(a_buf[r, sl] + b_buf[r, sl], 0.0)
            return 0
        lax.fori_loop(0, CH, _row, 0)

        pltpu.sync_copy(m_buf, acc.at[idx_d], add=True)
        return 0
    lax.fori_loop(0, NCHUNK, _chunk, 0)

    plsc.subcore_barrier()

    for k in range(RPT // CH):
        off = rbase + k * CH
        pltpu.sync_copy(acc.at[pl.ds(off, CH), :],
                        out_hbm.at[c, pl.ds(off, CH), :])


def _tc_post_body(h_ref, p_ref, cnt_ref, wnt_ref, wnb_ref, bn_ref, g_ref,
                  be_ref, wo_ref, bo_ref, o_ref):
    h = h_ref[...]
    psum = p_ref[0] + p_ref[1]
    iq = lax.broadcasted_iota(jnp.int32, (NPAD, HQ), 0)
    qq = lax.broadcasted_iota(jnp.int32, (NPAD, HQ), 1)
    oq = (iq // 128 == qq).astype(jnp.float32)
    ir = lax.broadcasted_iota(jnp.int32, (NPAD, 128), 0)
    rr = lax.broadcasted_iota(jnp.int32, (NPAD, 128), 1)
    orr = (ir % 128 == rr).astype(jnp.float32)
    tmp = jnp.dot(oq, cnt_ref[...], preferred_element_type=jnp.float32)
    cnt = jnp.sum(tmp * orr, axis=1, keepdims=True)
    pooled = (psum / jnp.maximum(cnt, 1.0))[:N, :]
    nxt = jnp.dot(h, wnt_ref[...], preferred_element_type=jnp.float32)
    nxt = nxt + jnp.dot(pooled, wnb_ref[...],
                        preferred_element_type=jnp.float32)
    nxt = jnp.maximum(nxt + bn_ref[...], 0.0)
    mu = jnp.mean(nxt, axis=-1, keepdims=True)
    var = jnp.mean((nxt - mu) * (nxt - mu), axis=-1, keepdims=True)
    ln = g_ref[...] * (nxt - mu) / jnp.sqrt(var + 1e-5) + be_ref[...]
    new_h = h + ln
    agg = jnp.sum(new_h, axis=0, keepdims=True) * (1.0 / N)
    o_ref[...] = jnp.dot(agg, wo_ref[...],
                         preferred_element_type=jnp.float32) + bo_ref[...]


def _tc_post(h, partials, counts, wn_top, wn_bot, b_next, gamma, beta,
             w_out, b_out):
    return pl.pallas_call(
        _tc_post_body,
        out_shape=jax.ShapeDtypeStruct((1, OUT), jnp.float32),
    )(h, partials, counts, wn_top, wn_bot, b_next.reshape(1, H),
      gamma.reshape(1, H), beta.reshape(1, H), w_out, b_out.reshape(1, OUT))


@jax.jit
def kernel(x, edge_index, W_proj, b_proj, W_msg, b_msg, W_next, b_next,
           gamma, beta, W_out, b_out):
    src = edge_index[0]
    dst = edge_index[1]
    h, a, b, counts = _tc_pre(x, W_proj, b_proj, W_msg[:H], W_msg[H:],
                              b_msg, dst)
    partials = _sc_edge(a, b, src, dst)
    return _tc_post(h, partials, counts, W_next[:H], W_next[H:], b_next,
                    gamma, beta, W_out, b_out)
